# Initial kernel scaffold; baseline (speedup 1.0000x reference)
import jax
import jax.numpy as jnp
from jax.experimental import pallas as pl


def kernel(inputs, W):
    raise NotImplementedError("write your pallas kernel here")


# speedup vs baseline: 33.9552x; 33.9552x over previous
import functools

import jax
import jax.numpy as jnp
from jax.experimental import pallas as pl

_BBLK = 128


def _body(x_ref, w_ref, o_ref, *, C, H, WID, U):
    acc = jnp.zeros((x_ref.shape[0], U), dtype=jnp.float32)
    for c in range(C):
        for h in range(H):
            acc = acc + jnp.dot(
                x_ref[:, c, h, :], w_ref[c, h, :, :],
                preferred_element_type=jnp.float32)
    line = jax.nn.sigmoid(acc)

    l = [line[:, h:h + 1] for h in range(H)]

    def cs(a, b):
        return jnp.minimum(a, b), jnp.maximum(a, b)

    a0, a1 = cs(l[0], l[1])
    a2, a3 = cs(l[2], l[3])
    b0, b2 = cs(a0, a2)
    b1, b3 = cs(a1, a3)
    c1, c2 = cs(b1, b2)
    med = (c1 + c2) * 0.5

    m = [(med < l[h]) for h in range(H)]
    mt = [mm.astype(jnp.int32) for mm in m]
    mf = [1 - v for v in mt]
    ct, cf = [], []
    st = sf = 0
    for h in range(H):
        st = st + mt[h]
        sf = sf + mf[h]
        ct.append(st)
        cf.append(sf)
    n_t, n_f = ct[-1], cf[-1]
    rank_a = [jnp.where(m[h], ct[h] - 1, n_t + cf[h] - 1) for h in range(H)]
    rank_d = [jnp.where(m[h], n_f + ct[h] - 1, cf[h] - 1) for h in range(H)]

    for s in range(2):
        coef = [
            (rank_a[h] == s).astype(jnp.float32)
            + (rank_d[h] == s).astype(jnp.float32) * 0.0001
            for h in range(H)
        ]
        for c in range(C):
            r = coef[0] * x_ref[:, c, 0, :]
            for h in range(1, H):
                r = r + coef[h] * x_ref[:, c, h, :]
            o_ref[:, c, s, :] = r


def kernel(inputs, W):
    B, C, H, WID = inputs.shape
    U = W.shape[1]
    Wr = W.reshape(H, WID, C, U).transpose(2, 0, 1, 3)
    body = functools.partial(_body, C=C, H=H, WID=WID, U=U)
    return pl.pallas_call(
        body,
        grid=(B // _BBLK,),
        in_specs=[
            pl.BlockSpec((_BBLK, C, H, WID), lambda i: (i, 0, 0, 0)),
            pl.BlockSpec((C, H, WID, U), lambda i: (0, 0, 0, 0)),
        ],
        out_specs=pl.BlockSpec((_BBLK, C, 2, WID), lambda i: (i, 0, 0, 0)),
        out_shape=jax.ShapeDtypeStruct((B, C, 2, WID), jnp.float32),
    )(inputs, Wr)
